# baseline (speedup 1.0000x reference)
import jax
import jax.numpy as jnp
from jax.experimental import pallas as pl

SCALING = 32.0 / 64.0


def _fused_kernel(x_ref, wcat_ref, a_ref, r_ref, s_ref, t_ref,
                  bias_ref, out_ref):
    xb = x_ref[...].astype(jnp.bfloat16)
    h = jnp.dot(xb, a_ref[...], preferred_element_type=jnp.float32)
    h = jnp.maximum(h, 0.0)
    logits = jnp.dot(xb, r_ref[...], preferred_element_type=jnp.float32)
    m = jnp.max(logits, axis=-1, keepdims=True)
    p = jnp.exp(logits - m)
    wts = p / jnp.sum(p, axis=-1, keepdims=True)
    w_rep = jnp.dot(wts.astype(jnp.bfloat16), s_ref[...],
                    preferred_element_type=jnp.float32)
    h_tile = jnp.dot(h.astype(jnp.bfloat16), t_ref[...],
                     preferred_element_type=jnp.float32)
    hw = (w_rep * h_tile).astype(jnp.bfloat16)
    xcat = jnp.concatenate([xb, hw], axis=1)
    acc = jnp.dot(xcat, wcat_ref[...], preferred_element_type=jnp.float32)
    out_ref[...] = acc + bias_ref[...]


def kernel(x, base_W, base_b, A, B, router_W):
    n, d_in = x.shape
    d_out = base_W.shape[0]
    e, _, r = B.shape
    tn = 512 if n % 512 == 0 else n

    w_t = base_W.T.astype(jnp.bfloat16)
    a_t = A.T.astype(jnp.bfloat16)
    r_t = router_W.T.astype(jnp.bfloat16)
    b_flat = (SCALING * B.transpose(0, 2, 1).reshape(e * r, d_out)
              ).astype(jnp.bfloat16)
    w_cat = jnp.concatenate([w_t, b_flat], axis=0)
    bias = base_b.reshape(1, d_out)
    j = jnp.arange(e * r)
    s_pat = (j // r == jnp.arange(e)[:, None]).astype(jnp.bfloat16)
    t_pat = (j % r == jnp.arange(r)[:, None]).astype(jnp.bfloat16)

    return pl.pallas_call(
        _fused_kernel,
        grid=(n // tn,),
        in_specs=[
            pl.BlockSpec((tn, d_in), lambda i: (i, 0)),
            pl.BlockSpec((d_in + e * r, d_out), lambda i: (0, 0)),
            pl.BlockSpec((d_in, r), lambda i: (0, 0)),
            pl.BlockSpec((d_in, e), lambda i: (0, 0)),
            pl.BlockSpec((e, e * r), lambda i: (0, 0)),
            pl.BlockSpec((r, e * r), lambda i: (0, 0)),
            pl.BlockSpec((1, d_out), lambda i: (0, 0)),
        ],
        out_specs=pl.BlockSpec((tn, d_out), lambda i: (i, 0)),
        out_shape=jax.ShapeDtypeStruct((n, d_out), jnp.float32),
    )(x, w_cat, a_t, r_t, s_pat, t_pat, bias)


# speedup vs baseline: 1.0268x; 1.0268x over previous
import jax
import jax.numpy as jnp
from jax.experimental import pallas as pl

SCALING = 32.0 / 64.0


def _fused_kernel(x_ref, wbig_ref, bflat_ref, s_ref, t_ref, bias_ref,
                  out_ref):
    d_out = out_ref.shape[1]
    r = t_ref.shape[0]
    e = s_ref.shape[0]
    a_off = d_out
    r_off = d_out + ((r + 127) // 128) * 128

    xb = x_ref[...].astype(jnp.bfloat16)
    y = jnp.dot(xb, wbig_ref[...], preferred_element_type=jnp.float32)
    h = jnp.maximum(y[:, a_off:a_off + r], 0.0)
    logits = y[:, r_off:r_off + e]
    m = jnp.max(logits, axis=-1, keepdims=True)
    p = jnp.exp(logits - m)
    wts = p / jnp.sum(p, axis=-1, keepdims=True)
    w_rep = jnp.dot(wts.astype(jnp.bfloat16), s_ref[...],
                    preferred_element_type=jnp.float32)
    h_tile = jnp.dot(h.astype(jnp.bfloat16), t_ref[...],
                     preferred_element_type=jnp.float32)
    hw = (w_rep * h_tile).astype(jnp.bfloat16)
    delta = jnp.dot(hw, bflat_ref[...], preferred_element_type=jnp.float32)
    out_ref[...] = y[:, :d_out] + delta + bias_ref[...]


def kernel(x, base_W, base_b, A, B, router_W):
    n, d_in = x.shape
    d_out = base_W.shape[0]
    e, _, r = B.shape
    tn = 512 if n % 512 == 0 else n
    rpad = ((r + 127) // 128) * 128
    epad = ((e + 127) // 128) * 128

    w_t = base_W.T.astype(jnp.bfloat16)
    a_t = A.T.astype(jnp.bfloat16)
    r_t = router_W.T.astype(jnp.bfloat16)
    a_pad = jnp.pad(a_t, ((0, 0), (0, rpad - r)))
    r_pad = jnp.pad(r_t, ((0, 0), (0, epad - e)))
    w_big = jnp.concatenate([w_t, a_pad, r_pad], axis=1)
    b_flat = (SCALING * B.transpose(0, 2, 1).reshape(e * r, d_out)
              ).astype(jnp.bfloat16)
    bias = base_b.reshape(1, d_out)
    j = jnp.arange(e * r)
    s_pat = (j // r == jnp.arange(e)[:, None]).astype(jnp.bfloat16)
    t_pat = (j % r == jnp.arange(r)[:, None]).astype(jnp.bfloat16)

    n_big = d_out + rpad + epad
    return pl.pallas_call(
        _fused_kernel,
        grid=(n // tn,),
        in_specs=[
            pl.BlockSpec((tn, d_in), lambda i: (i, 0)),
            pl.BlockSpec((d_in, n_big), lambda i: (0, 0)),
            pl.BlockSpec((e * r, d_out), lambda i: (0, 0)),
            pl.BlockSpec((e, e * r), lambda i: (0, 0)),
            pl.BlockSpec((r, e * r), lambda i: (0, 0)),
            pl.BlockSpec((1, d_out), lambda i: (0, 0)),
        ],
        out_specs=pl.BlockSpec((tn, d_out), lambda i: (i, 0)),
        out_shape=jax.ShapeDtypeStruct((n, d_out), jnp.float32),
    )(x, w_big, b_flat, s_pat, t_pat, bias)


# speedup vs baseline: 1.0581x; 1.0305x over previous
import jax
import jax.numpy as jnp
from jax.experimental import pallas as pl
from jax.experimental.pallas import tpu as pltpu

SCALING = 32.0 / 64.0


def _fused_kernel(x_ref, wbig_ref, bflat_ref, s_ref, t_ref, bias_ref,
                  out_ref, y_ref, *, nblocks):
    d_out = out_ref.shape[1]
    r = t_ref.shape[0]
    e = s_ref.shape[0]
    a_off = d_out
    r_off = d_out + ((r + 127) // 128) * 128
    i = pl.program_id(0)

    @pl.when(i < nblocks)
    def _produce():
        xb = x_ref[...].astype(jnp.bfloat16)
        y_ref[i % 2] = jnp.dot(xb, wbig_ref[...],
                               preferred_element_type=jnp.float32)

    @pl.when(i > 0)
    def _consume():
        y = y_ref[(i - 1) % 2]
        h = jnp.maximum(y[:, a_off:a_off + r], 0.0)
        logits = y[:, r_off:r_off + e]
        m = jnp.max(logits, axis=-1, keepdims=True)
        p = jnp.exp(logits - m)
        wts = p / jnp.sum(p, axis=-1, keepdims=True)
        w_rep = jnp.dot(wts.astype(jnp.bfloat16), s_ref[...],
                        preferred_element_type=jnp.float32)
        h_tile = jnp.dot(h.astype(jnp.bfloat16), t_ref[...],
                         preferred_element_type=jnp.float32)
        hw = (w_rep * h_tile).astype(jnp.bfloat16)
        delta = jnp.dot(hw, bflat_ref[...],
                        preferred_element_type=jnp.float32)
        out_ref[...] = y[:, :d_out] + delta + bias_ref[...]


def kernel(x, base_W, base_b, A, B, router_W):
    import functools
    n, d_in = x.shape
    d_out = base_W.shape[0]
    e, _, r = B.shape
    tn = 512 if n % 512 == 0 else n
    nblocks = n // tn
    rpad = ((r + 127) // 128) * 128
    epad = ((e + 127) // 128) * 128

    w_t = base_W.T.astype(jnp.bfloat16)
    a_t = A.T.astype(jnp.bfloat16)
    r_t = router_W.T.astype(jnp.bfloat16)
    a_pad = jnp.pad(a_t, ((0, 0), (0, rpad - r)))
    r_pad = jnp.pad(r_t, ((0, 0), (0, epad - e)))
    w_big = jnp.concatenate([w_t, a_pad, r_pad], axis=1)
    b_flat = (SCALING * B.transpose(0, 2, 1).reshape(e * r, d_out)
              ).astype(jnp.bfloat16)
    bias = base_b.reshape(1, d_out)
    j = jnp.arange(e * r)
    s_pat = (j // r == jnp.arange(e)[:, None]).astype(jnp.bfloat16)
    t_pat = (j % r == jnp.arange(r)[:, None]).astype(jnp.bfloat16)

    n_big = d_out + rpad + epad
    last = nblocks - 1
    return pl.pallas_call(
        functools.partial(_fused_kernel, nblocks=nblocks),
        grid=(nblocks + 1,),
        in_specs=[
            pl.BlockSpec((tn, d_in), lambda i: (jnp.minimum(i, last), 0)),
            pl.BlockSpec((d_in, n_big), lambda i: (0, 0)),
            pl.BlockSpec((e * r, d_out), lambda i: (0, 0)),
            pl.BlockSpec((e, e * r), lambda i: (0, 0)),
            pl.BlockSpec((r, e * r), lambda i: (0, 0)),
            pl.BlockSpec((1, d_out), lambda i: (0, 0)),
        ],
        out_specs=pl.BlockSpec((tn, d_out),
                               lambda i: (jnp.maximum(i - 1, 0), 0)),
        out_shape=jax.ShapeDtypeStruct((n, d_out), jnp.float32),
        scratch_shapes=[pltpu.VMEM((2, tn, n_big), jnp.float32)],
    )(x, w_big, b_flat, s_pat, t_pat, bias)


# speedup vs baseline: 1.0625x; 1.0041x over previous
import jax
import jax.numpy as jnp
from jax.experimental import pallas as pl
from jax.experimental.pallas import tpu as pltpu

SCALING = 32.0 / 64.0


def _fused_kernel(x_ref, wbig_ref, bflat_ref, s_ref, t_ref, bias_ref,
                  out_ref, y_ref, *, nblocks):
    d_out = out_ref.shape[1]
    r = t_ref.shape[0]
    e = s_ref.shape[0]
    r_off = d_out
    a_off = d_out + e
    i = pl.program_id(0)

    @pl.when(i < nblocks)
    def _produce():
        xb = x_ref[...].astype(jnp.bfloat16)
        y_ref[i % 2] = jnp.dot(xb, wbig_ref[...],
                               preferred_element_type=jnp.float32)

    @pl.when(i > 0)
    def _consume():
        y = y_ref[(i - 1) % 2]
        h = jnp.maximum(y[:, a_off:a_off + r], 0.0)
        logits = y[:, r_off:r_off + e]
        m = jnp.max(logits, axis=-1, keepdims=True)
        p = jnp.exp(logits - m)
        wts = p / jnp.sum(p, axis=-1, keepdims=True)
        w_rep = jnp.dot(wts.astype(jnp.bfloat16), s_ref[...],
                        preferred_element_type=jnp.float32)
        h_tile = jnp.dot(h.astype(jnp.bfloat16), t_ref[...],
                         preferred_element_type=jnp.float32)
        hw = (w_rep * h_tile).astype(jnp.bfloat16)
        delta = jnp.dot(hw, bflat_ref[...],
                        preferred_element_type=jnp.float32)
        out_ref[...] = y[:, :d_out] + delta + bias_ref[...]


def kernel(x, base_W, base_b, A, B, router_W):
    import functools
    n, d_in = x.shape
    d_out = base_W.shape[0]
    e, _, r = B.shape
    tn = 512 if n % 512 == 0 else n
    nblocks = n // tn
    extra = ((e + r + 127) // 128) * 128

    w_t = base_W.T.astype(jnp.bfloat16)
    a_t = A.T.astype(jnp.bfloat16)
    r_t = router_W.T.astype(jnp.bfloat16)
    tail = jnp.pad(jnp.concatenate([r_t, a_t], axis=1),
                   ((0, 0), (0, extra - e - r)))
    w_big = jnp.concatenate([w_t, tail], axis=1)
    b_flat = (SCALING * B.transpose(0, 2, 1).reshape(e * r, d_out)
              ).astype(jnp.bfloat16)
    bias = base_b.reshape(1, d_out)
    j = jnp.arange(e * r)
    s_pat = (j // r == jnp.arange(e)[:, None]).astype(jnp.bfloat16)
    t_pat = (j % r == jnp.arange(r)[:, None]).astype(jnp.bfloat16)

    n_big = d_out + extra
    last = nblocks - 1
    return pl.pallas_call(
        functools.partial(_fused_kernel, nblocks=nblocks),
        grid=(nblocks + 1,),
        in_specs=[
            pl.BlockSpec((tn, d_in), lambda i: (jnp.minimum(i, last), 0)),
            pl.BlockSpec((d_in, n_big), lambda i: (0, 0)),
            pl.BlockSpec((e * r, d_out), lambda i: (0, 0)),
            pl.BlockSpec((e, e * r), lambda i: (0, 0)),
            pl.BlockSpec((r, e * r), lambda i: (0, 0)),
            pl.BlockSpec((1, d_out), lambda i: (0, 0)),
        ],
        out_specs=pl.BlockSpec((tn, d_out),
                               lambda i: (jnp.maximum(i - 1, 0), 0)),
        out_shape=jax.ShapeDtypeStruct((n, d_out), jnp.float32),
        scratch_shapes=[pltpu.VMEM((2, tn, n_big), jnp.float32)],
    )(x, w_big, b_flat, s_pat, t_pat, bias)
